# baseline (speedup 1.0000x reference)
import jax
import jax.numpy as jnp
from jax import lax
from jax.experimental import pallas as pl
from jax.experimental.pallas import tpu as pltpu
from jax.experimental.pallas import tpu_sc as plsc

VOCAB = 100000
HIDDEN = 128
BATCH = 4096
SEQ = 50
SEQ_PAD = 56
EPS = 1e-3

NUM_WORKERS = 32
B_SC = 2560
B_TC = BATCH - B_SC
ROWS_PER_WORKER = B_SC // NUM_WORKERS
LANES = 16
NCHUNK = HIDDEN // LANES
TC_CHUNK = 128


PAIR = 2 * SEQ_PAD


def _sc_pool_body(idx_hbm, table_hbm, out_hbm, idx_v, g0, g1, pooled, sem0, sem1):
    nc = 2
    wid = lax.axis_index("s") * nc + lax.axis_index("c")
    npair = ROWS_PER_WORKER // 2
    base = wid * npair

    pltpu.sync_copy(idx_hbm.at[pl.ds(base, npair)], idx_v)

    def gather(p, buf, sem):
        return pltpu.make_async_copy(
            table_hbm.at[idx_v.at[p, pl.ds(0, PAIR)]], buf, sem
        )

    def accumulate(buf, p):
        for h in range(2):
            b = 2 * p + h
            off = SEQ_PAD * h
            accs = [buf[off, pl.ds(c * LANES, LANES)] for c in range(NCHUNK)]
            for l in range(1, SEQ):
                for c in range(NCHUNK):
                    accs[c] = accs[c] + buf[off + l, pl.ds(c * LANES, LANES)]
            for c in range(NCHUNK):
                pooled[b, pl.ds(c * LANES, LANES)] = accs[c] * (1.0 / SEQ)

    gather(0, g0, sem0).start()

    def loop_body(i, _):
        p = 2 * i
        gather(p, g0, sem0).wait()
        gather(p + 1, g1, sem1).start()
        accumulate(g0, p)
        gather(p + 1, g1, sem1).wait()
        nxt = jnp.minimum(p + 2, npair - 1)
        gather(nxt, g0, sem0).start()
        accumulate(g1, p + 1)
        return _

    lax.fori_loop(0, npair // 2, loop_body, None)
    gather(ROWS_PER_WORKER // 2 - 1, g0, sem0).wait()

    pltpu.sync_copy(pooled, out_hbm.at[pl.ds(wid * ROWS_PER_WORKER, ROWS_PER_WORKER)])


@jax.jit
def _sc_pool(idx_padded, table):
    mesh = plsc.VectorSubcoreMesh(core_axis_name="c", subcore_axis_name="s")
    return pl.kernel(
        _sc_pool_body,
        mesh=mesh,
        out_type=jax.ShapeDtypeStruct((B_SC, HIDDEN), jnp.float32),
        scratch_types=[
            pltpu.VMEM((ROWS_PER_WORKER // 2, PAIR), jnp.int32),
            pltpu.VMEM((PAIR, HIDDEN), jnp.float32),
            pltpu.VMEM((PAIR, HIDDEN), jnp.float32),
            pltpu.VMEM((ROWS_PER_WORKER, HIDDEN), jnp.float32),
            pltpu.SemaphoreType.DMA,
            pltpu.SemaphoreType.DMA,
        ],
    )(idx_padded, table)


def _tc_pool_body(idx_ref, table_hbm, out_ref, tvmem, sem):
    @pl.when(pl.program_id(0) == 0)
    def _():
        pltpu.make_async_copy(table_hbm, tvmem, sem).start()
        pltpu.make_async_copy(table_hbm, tvmem, sem).wait()

    for r in range(TC_CHUNK):
        acc = tvmem[pl.ds(idx_ref[r, 0], 1), :]
        for l in range(1, SEQ):
            acc = acc + tvmem[pl.ds(idx_ref[r, l], 1), :]
        out_ref[pl.ds(r, 1), :] = acc * (1.0 / SEQ)


@jax.jit
def _tc_pool(idx, table):
    return pl.pallas_call(
        _tc_pool_body,
        grid=(B_TC // TC_CHUNK,),
        in_specs=[
            pl.BlockSpec((TC_CHUNK, SEQ), lambda i: (i, 0), memory_space=pltpu.SMEM),
            pl.BlockSpec(memory_space=pl.ANY),
        ],
        out_specs=pl.BlockSpec((TC_CHUNK, HIDDEN), lambda i: (i, 0)),
        out_shape=jax.ShapeDtypeStruct((B_TC, HIDDEN), jnp.float32),
        scratch_shapes=[
            pltpu.VMEM((VOCAB, HIDDEN), jnp.float32),
            pltpu.SemaphoreType.DMA,
        ],
    )(idx, table)


def _tc_layernorm_body(a_ref, b_ref, gamma_ref, beta_ref, o_ref):
    a = a_ref[:, :]
    b = b_ref[:, :]
    mu = (jnp.sum(a, axis=0, keepdims=True) + jnp.sum(b, axis=0, keepdims=True)) * (
        1.0 / BATCH
    )
    da = a - mu
    db = b - mu
    var = (
        jnp.sum(da * da, axis=0, keepdims=True)
        + jnp.sum(db * db, axis=0, keepdims=True)
    ) * (1.0 / BATCH)
    r = lax.rsqrt(var + EPS)
    o_ref[pl.ds(0, B_TC), :] = (
        da * r * gamma_ref[pl.ds(0, B_TC), :] + beta_ref[pl.ds(0, B_TC), :]
    )
    o_ref[pl.ds(B_TC, B_SC), :] = (
        db * r * gamma_ref[pl.ds(B_TC, B_SC), :] + beta_ref[pl.ds(B_TC, B_SC), :]
    )


@jax.jit
def _tc_layernorm(pooled_tc, pooled_sc, gamma, beta):
    return pl.pallas_call(
        _tc_layernorm_body,
        out_shape=jax.ShapeDtypeStruct((BATCH, HIDDEN), jnp.float32),
    )(
        pooled_tc,
        pooled_sc,
        gamma.reshape(BATCH, 1),
        beta.reshape(BATCH, 1),
    )


def kernel(inputs, table, bias, gamma, beta):
    del bias
    idx_sc = jnp.concatenate(
        [inputs[B_TC:], jnp.zeros((B_SC, SEQ_PAD - SEQ), jnp.int32)], axis=1
    ).reshape(B_SC // 2, PAIR)
    pooled_sc = _sc_pool(idx_sc, table)
    pooled_tc = _tc_pool(inputs, table)
    return _tc_layernorm(pooled_tc, pooled_sc, gamma, beta)


# speedup vs baseline: 6.1104x; 6.1104x over previous
import jax
import jax.numpy as jnp
from jax import lax
from jax.experimental import pallas as pl
from jax.experimental.pallas import tpu as pltpu
from jax.experimental.pallas import tpu_sc as plsc

VOCAB = 100000
HIDDEN = 128
BATCH = 4096
SEQ = 50
SEQ_PAD = 56
EPS = 1e-3

NUM_WORKERS = 32
B_SC = 2560
B_TC = BATCH - B_SC
ROWS_PER_WORKER = B_SC // NUM_WORKERS
LANES = 16
NCHUNK = HIDDEN // LANES
TC_CHUNK = 128


def _sc_pool_body(idx_hbm, table_hbm, out_hbm, idx_v, g0, g1, pooled, sem0, sem1):
    nc = 2
    wid = lax.axis_index("s") * nc + lax.axis_index("c")
    base = wid * ROWS_PER_WORKER

    pltpu.sync_copy(idx_hbm.at[pl.ds(base, ROWS_PER_WORKER)], idx_v)

    def gather(b, buf, sem):
        return pltpu.make_async_copy(
            table_hbm.at[idx_v.at[b, pl.ds(0, SEQ)]], buf, sem
        )

    def accumulate(buf, b):
        accs = [buf[0, pl.ds(c * LANES, LANES)] for c in range(NCHUNK)]
        for l in range(1, SEQ):
            for c in range(NCHUNK):
                accs[c] = accs[c] + buf[l, pl.ds(c * LANES, LANES)]
        for c in range(NCHUNK):
            pooled[b, pl.ds(c * LANES, LANES)] = accs[c] * (1.0 / SEQ)

    gather(0, g0, sem0).start()

    def loop_body(i, _):
        b = 2 * i
        gather(b, g0, sem0).wait()
        gather(b + 1, g1, sem1).start()
        accumulate(g0, b)
        gather(b + 1, g1, sem1).wait()
        nxt = jnp.minimum(b + 2, ROWS_PER_WORKER - 1)
        gather(nxt, g0, sem0).start()
        accumulate(g1, b + 1)
        return _

    lax.fori_loop(0, ROWS_PER_WORKER // 2, loop_body, None)
    gather(ROWS_PER_WORKER - 1, g0, sem0).wait()

    pltpu.sync_copy(pooled, out_hbm.at[pl.ds(base, ROWS_PER_WORKER)])


@jax.jit
def _sc_pool(idx_padded, table):
    mesh = plsc.VectorSubcoreMesh(core_axis_name="c", subcore_axis_name="s")
    return pl.kernel(
        _sc_pool_body,
        mesh=mesh,
        out_type=jax.ShapeDtypeStruct((B_SC, HIDDEN), jnp.float32),
        scratch_types=[
            pltpu.VMEM((ROWS_PER_WORKER, SEQ_PAD), jnp.int32),
            pltpu.VMEM((SEQ, HIDDEN), jnp.float32),
            pltpu.VMEM((SEQ, HIDDEN), jnp.float32),
            pltpu.VMEM((ROWS_PER_WORKER, HIDDEN), jnp.float32),
            pltpu.SemaphoreType.DMA,
            pltpu.SemaphoreType.DMA,
        ],
    )(idx_padded, table)


N_COPY = 4
COPY_ROWS = VOCAB // N_COPY


def _tc_pool_body(idx_ref, table_hbm, out_ref, tvmem, *sems):
    @pl.when(pl.program_id(0) == 0)
    def _():
        for k in range(N_COPY):
            pltpu.make_async_copy(
                table_hbm.at[pl.ds(k * COPY_ROWS, COPY_ROWS)],
                tvmem.at[pl.ds(k * COPY_ROWS, COPY_ROWS)],
                sems[k],
            ).start()
        for k in range(N_COPY):
            pltpu.make_async_copy(
                table_hbm.at[pl.ds(k * COPY_ROWS, COPY_ROWS)],
                tvmem.at[pl.ds(k * COPY_ROWS, COPY_ROWS)],
                sems[k],
            ).wait()

    for r in range(TC_CHUNK):
        acc = tvmem[pl.ds(idx_ref[r, 0], 1), :]
        for l in range(1, SEQ):
            acc = acc + tvmem[pl.ds(idx_ref[r, l], 1), :]
        out_ref[pl.ds(r, 1), :] = acc * (1.0 / SEQ)


@jax.jit
def _tc_pool(idx, table):
    return pl.pallas_call(
        _tc_pool_body,
        grid=(B_TC // TC_CHUNK,),
        in_specs=[
            pl.BlockSpec((TC_CHUNK, SEQ), lambda i: (i, 0), memory_space=pltpu.SMEM),
            pl.BlockSpec(memory_space=pl.ANY),
        ],
        out_specs=pl.BlockSpec((TC_CHUNK, HIDDEN), lambda i: (i, 0)),
        out_shape=jax.ShapeDtypeStruct((B_TC, HIDDEN), jnp.float32),
        scratch_shapes=[pltpu.VMEM((VOCAB, HIDDEN), jnp.float32)]
        + [pltpu.SemaphoreType.DMA for _ in range(N_COPY)],
    )(idx, table)


def _tc_layernorm_body(a_ref, b_ref, gamma_ref, beta_ref, o_ref):
    a = a_ref[:, :]
    b = b_ref[:, :]
    mu = (jnp.sum(a, axis=0, keepdims=True) + jnp.sum(b, axis=0, keepdims=True)) * (
        1.0 / BATCH
    )
    da = a - mu
    db = b - mu
    var = (
        jnp.sum(da * da, axis=0, keepdims=True)
        + jnp.sum(db * db, axis=0, keepdims=True)
    ) * (1.0 / BATCH)
    r = lax.rsqrt(var + EPS)
    o_ref[pl.ds(0, B_TC), :] = (
        da * r * gamma_ref[pl.ds(0, B_TC), :] + beta_ref[pl.ds(0, B_TC), :]
    )
    o_ref[pl.ds(B_TC, B_SC), :] = (
        db * r * gamma_ref[pl.ds(B_TC, B_SC), :] + beta_ref[pl.ds(B_TC, B_SC), :]
    )


@jax.jit
def _tc_layernorm(pooled_tc, pooled_sc, gamma, beta):
    return pl.pallas_call(
        _tc_layernorm_body,
        out_shape=jax.ShapeDtypeStruct((BATCH, HIDDEN), jnp.float32),
    )(
        pooled_tc,
        pooled_sc,
        gamma.reshape(BATCH, 1),
        beta.reshape(BATCH, 1),
    )


def kernel(inputs, table, bias, gamma, beta):
    del bias
    idx_sc = jnp.concatenate(
        [inputs[B_TC:], jnp.zeros((B_SC, SEQ_PAD - SEQ), jnp.int32)], axis=1
    )
    pooled_sc = _sc_pool(idx_sc, table)
    pooled_tc = _tc_pool(inputs, table)
    return _tc_layernorm(pooled_tc, pooled_sc, gamma, beta)


# speedup vs baseline: 6.1147x; 1.0007x over previous
import jax
import jax.numpy as jnp
from jax import lax
from jax.experimental import pallas as pl
from jax.experimental.pallas import tpu as pltpu
from jax.experimental.pallas import tpu_sc as plsc

VOCAB = 100000
HIDDEN = 128
BATCH = 4096
SEQ = 50
SEQ_PAD = 56
EPS = 1e-3

NUM_WORKERS = 32
B_SC = 2560
B_TC = BATCH - B_SC
ROWS_PER_WORKER = B_SC // NUM_WORKERS
LANES = 16
NCHUNK = HIDDEN // LANES
TC_CHUNK = 256


def _sc_pool_body(idx_hbm, table_hbm, out_hbm, idx_v, g0, g1, pooled, sem0, sem1):
    nc = 2
    wid = lax.axis_index("s") * nc + lax.axis_index("c")
    base = wid * ROWS_PER_WORKER

    pltpu.sync_copy(idx_hbm.at[pl.ds(base, ROWS_PER_WORKER)], idx_v)

    def gather(b, buf, sem):
        return pltpu.make_async_copy(
            table_hbm.at[idx_v.at[b, pl.ds(0, SEQ)]], buf, sem
        )

    def accumulate(buf, b):
        accs = [buf[0, pl.ds(c * LANES, LANES)] for c in range(NCHUNK)]
        for l in range(1, SEQ):
            for c in range(NCHUNK):
                accs[c] = accs[c] + buf[l, pl.ds(c * LANES, LANES)]
        for c in range(NCHUNK):
            pooled[b, pl.ds(c * LANES, LANES)] = accs[c] * (1.0 / SEQ)

    gather(0, g0, sem0).start()

    def loop_body(i, _):
        b = 2 * i
        gather(b, g0, sem0).wait()
        gather(b + 1, g1, sem1).start()
        accumulate(g0, b)
        gather(b + 1, g1, sem1).wait()
        nxt = jnp.minimum(b + 2, ROWS_PER_WORKER - 1)
        gather(nxt, g0, sem0).start()
        accumulate(g1, b + 1)
        return _

    lax.fori_loop(0, ROWS_PER_WORKER // 2, loop_body, None)
    gather(ROWS_PER_WORKER - 1, g0, sem0).wait()

    pltpu.sync_copy(pooled, out_hbm.at[pl.ds(base, ROWS_PER_WORKER)])


@jax.jit
def _sc_pool(idx_padded, table):
    mesh = plsc.VectorSubcoreMesh(core_axis_name="c", subcore_axis_name="s")
    return pl.kernel(
        _sc_pool_body,
        mesh=mesh,
        out_type=jax.ShapeDtypeStruct((B_SC, HIDDEN), jnp.float32),
        scratch_types=[
            pltpu.VMEM((ROWS_PER_WORKER, SEQ_PAD), jnp.int32),
            pltpu.VMEM((SEQ, HIDDEN), jnp.float32),
            pltpu.VMEM((SEQ, HIDDEN), jnp.float32),
            pltpu.VMEM((ROWS_PER_WORKER, HIDDEN), jnp.float32),
            pltpu.SemaphoreType.DMA,
            pltpu.SemaphoreType.DMA,
        ],
    )(idx_padded, table)


N_COPY = 4
COPY_ROWS = VOCAB // N_COPY


def _tc_pool_body(idx_ref, table_hbm, out_ref, tvmem, *sems):
    @pl.when(pl.program_id(0) == 0)
    def _():
        for k in range(N_COPY):
            pltpu.make_async_copy(
                table_hbm.at[pl.ds(k * COPY_ROWS, COPY_ROWS)],
                tvmem.at[pl.ds(k * COPY_ROWS, COPY_ROWS)],
                sems[k],
            ).start()
        for k in range(N_COPY):
            pltpu.make_async_copy(
                table_hbm.at[pl.ds(k * COPY_ROWS, COPY_ROWS)],
                tvmem.at[pl.ds(k * COPY_ROWS, COPY_ROWS)],
                sems[k],
            ).wait()

    for r in range(TC_CHUNK):
        acc = tvmem[pl.ds(idx_ref[r, 0], 1), :]
        for l in range(1, SEQ):
            acc = acc + tvmem[pl.ds(idx_ref[r, l], 1), :]
        out_ref[pl.ds(r, 1), :] = acc * (1.0 / SEQ)


@jax.jit
def _tc_pool(idx, table):
    return pl.pallas_call(
        _tc_pool_body,
        grid=(B_TC // TC_CHUNK,),
        in_specs=[
            pl.BlockSpec((TC_CHUNK, SEQ), lambda i: (i, 0), memory_space=pltpu.SMEM),
            pl.BlockSpec(memory_space=pl.ANY),
        ],
        out_specs=pl.BlockSpec((TC_CHUNK, HIDDEN), lambda i: (i, 0)),
        out_shape=jax.ShapeDtypeStruct((B_TC, HIDDEN), jnp.float32),
        scratch_shapes=[pltpu.VMEM((VOCAB, HIDDEN), jnp.float32)]
        + [pltpu.SemaphoreType.DMA for _ in range(N_COPY)],
    )(idx, table)


def _tc_layernorm_body(a_ref, b_ref, gamma_ref, beta_ref, o_ref):
    a = a_ref[:, :]
    b = b_ref[:, :]
    mu = (jnp.sum(a, axis=0, keepdims=True) + jnp.sum(b, axis=0, keepdims=True)) * (
        1.0 / BATCH
    )
    da = a - mu
    db = b - mu
    var = (
        jnp.sum(da * da, axis=0, keepdims=True)
        + jnp.sum(db * db, axis=0, keepdims=True)
    ) * (1.0 / BATCH)
    r = lax.rsqrt(var + EPS)
    o_ref[pl.ds(0, B_TC), :] = (
        da * r * gamma_ref[pl.ds(0, B_TC), :] + beta_ref[pl.ds(0, B_TC), :]
    )
    o_ref[pl.ds(B_TC, B_SC), :] = (
        db * r * gamma_ref[pl.ds(B_TC, B_SC), :] + beta_ref[pl.ds(B_TC, B_SC), :]
    )


@jax.jit
def _tc_layernorm(pooled_tc, pooled_sc, gamma, beta):
    return pl.pallas_call(
        _tc_layernorm_body,
        out_shape=jax.ShapeDtypeStruct((BATCH, HIDDEN), jnp.float32),
    )(
        pooled_tc,
        pooled_sc,
        gamma.reshape(BATCH, 1),
        beta.reshape(BATCH, 1),
    )


def kernel(inputs, table, bias, gamma, beta):
    del bias
    idx_sc = jnp.concatenate(
        [inputs[B_TC:], jnp.zeros((B_SC, SEQ_PAD - SEQ), jnp.int32)], axis=1
    )
    pooled_sc = _sc_pool(idx_sc, table)
    pooled_tc = _tc_pool(inputs, table)
    return _tc_layernorm(pooled_tc, pooled_sc, gamma, beta)


# speedup vs baseline: 6.1154x; 1.0001x over previous
import jax
import jax.numpy as jnp
from jax import lax
from jax.experimental import pallas as pl
from jax.experimental.pallas import tpu as pltpu
from jax.experimental.pallas import tpu_sc as plsc

VOCAB = 100000
HIDDEN = 128
BATCH = 4096
SEQ = 50
SEQ_PAD = 56
EPS = 1e-3

NUM_WORKERS = 32
B_SC = 2560
B_TC = BATCH - B_SC
ROWS_PER_WORKER = B_SC // NUM_WORKERS
LANES = 16
NCHUNK = HIDDEN // LANES
TC_CHUNK = 128


def _sc_pool_body(idx_hbm, table_hbm, out_hbm, idx_v, g0, g1, pooled, sem0, sem1):
    nc = 2
    wid = lax.axis_index("s") * nc + lax.axis_index("c")
    base = wid * ROWS_PER_WORKER

    pltpu.sync_copy(idx_hbm.at[pl.ds(base, ROWS_PER_WORKER)], idx_v)

    def gather(b, buf, sem):
        return pltpu.make_async_copy(
            table_hbm.at[idx_v.at[b, pl.ds(0, SEQ)]], buf, sem
        )

    def accumulate(buf, b):
        accs = [buf[0, pl.ds(c * LANES, LANES)] for c in range(NCHUNK)]
        for l in range(1, SEQ):
            for c in range(NCHUNK):
                accs[c] = accs[c] + buf[l, pl.ds(c * LANES, LANES)]
        for c in range(NCHUNK):
            pooled[b, pl.ds(c * LANES, LANES)] = accs[c] * (1.0 / SEQ)

    gather(0, g0, sem0).start()

    def loop_body(i, _):
        b = 2 * i
        gather(b, g0, sem0).wait()
        gather(b + 1, g1, sem1).start()
        accumulate(g0, b)
        gather(b + 1, g1, sem1).wait()
        nxt = jnp.minimum(b + 2, ROWS_PER_WORKER - 1)
        gather(nxt, g0, sem0).start()
        accumulate(g1, b + 1)
        return _

    lax.fori_loop(0, ROWS_PER_WORKER // 2, loop_body, None)
    gather(ROWS_PER_WORKER - 1, g0, sem0).wait()

    pltpu.sync_copy(pooled, out_hbm.at[pl.ds(base, ROWS_PER_WORKER)])


@jax.jit
def _sc_pool(idx_padded, table):
    mesh = plsc.VectorSubcoreMesh(core_axis_name="c", subcore_axis_name="s")
    return pl.kernel(
        _sc_pool_body,
        mesh=mesh,
        out_type=jax.ShapeDtypeStruct((B_SC, HIDDEN), jnp.float32),
        scratch_types=[
            pltpu.VMEM((ROWS_PER_WORKER, SEQ_PAD), jnp.int32),
            pltpu.VMEM((SEQ, HIDDEN), jnp.float32),
            pltpu.VMEM((SEQ, HIDDEN), jnp.float32),
            pltpu.VMEM((ROWS_PER_WORKER, HIDDEN), jnp.float32),
            pltpu.SemaphoreType.DMA,
            pltpu.SemaphoreType.DMA,
        ],
    )(idx_padded, table)


N_COPY = 4
COPY_ROWS = VOCAB // N_COPY


def _tc_pool_body(idx_ref, table_hbm, out_ref, tvmem, *sems):
    @pl.when(pl.program_id(0) == 0)
    def _():
        for k in range(N_COPY):
            pltpu.make_async_copy(
                table_hbm.at[pl.ds(k * COPY_ROWS, COPY_ROWS)],
                tvmem.at[pl.ds(k * COPY_ROWS, COPY_ROWS)],
                sems[k],
            ).start()
        for k in range(N_COPY):
            pltpu.make_async_copy(
                table_hbm.at[pl.ds(k * COPY_ROWS, COPY_ROWS)],
                tvmem.at[pl.ds(k * COPY_ROWS, COPY_ROWS)],
                sems[k],
            ).wait()

    for r in range(TC_CHUNK):
        acc = tvmem[pl.ds(idx_ref[r, 0], 1), :]
        for l in range(1, SEQ):
            acc = acc + tvmem[pl.ds(idx_ref[r, l], 1), :]
        out_ref[pl.ds(r, 1), :] = acc * (1.0 / SEQ)


@jax.jit
def _tc_pool(idx, table):
    return pl.pallas_call(
        _tc_pool_body,
        grid=(B_TC // TC_CHUNK,),
        in_specs=[
            pl.BlockSpec((TC_CHUNK, SEQ), lambda i: (i, 0), memory_space=pltpu.SMEM),
            pl.BlockSpec(memory_space=pl.ANY),
        ],
        out_specs=pl.BlockSpec((TC_CHUNK, HIDDEN), lambda i: (i, 0)),
        out_shape=jax.ShapeDtypeStruct((B_TC, HIDDEN), jnp.float32),
        scratch_shapes=[pltpu.VMEM((VOCAB, HIDDEN), jnp.float32)]
        + [pltpu.SemaphoreType.DMA for _ in range(N_COPY)],
    )(idx, table)


def _tc_layernorm_body(a_ref, b_ref, gamma_ref, beta_ref, o_ref):
    a = a_ref[:, :]
    b = b_ref[:, :]
    mu = (jnp.sum(a, axis=0, keepdims=True) + jnp.sum(b, axis=0, keepdims=True)) * (
        1.0 / BATCH
    )
    da = a - mu
    db = b - mu
    var = (
        jnp.sum(da * da, axis=0, keepdims=True)
        + jnp.sum(db * db, axis=0, keepdims=True)
    ) * (1.0 / BATCH)
    r = lax.rsqrt(var + EPS)
    o_ref[pl.ds(0, B_TC), :] = (
        da * r * gamma_ref[pl.ds(0, B_TC), :] + beta_ref[pl.ds(0, B_TC), :]
    )
    o_ref[pl.ds(B_TC, B_SC), :] = (
        db * r * gamma_ref[pl.ds(B_TC, B_SC), :] + beta_ref[pl.ds(B_TC, B_SC), :]
    )


@jax.jit
def _tc_layernorm(pooled_tc, pooled_sc, gamma, beta):
    return pl.pallas_call(
        _tc_layernorm_body,
        out_shape=jax.ShapeDtypeStruct((BATCH, HIDDEN), jnp.float32),
    )(
        pooled_tc,
        pooled_sc,
        gamma.reshape(BATCH, 1),
        beta.reshape(BATCH, 1),
    )


def kernel(inputs, table, bias, gamma, beta):
    del bias
    idx_sc = jnp.concatenate(
        [inputs[B_TC:], jnp.zeros((B_SC, SEQ_PAD - SEQ), jnp.int32)], axis=1
    )
    pooled_sc = _sc_pool(idx_sc, table)
    pooled_tc = _tc_pool(inputs, table)
    return _tc_layernorm(pooled_tc, pooled_sc, gamma, beta)


# speedup vs baseline: 6.1394x; 1.0039x over previous
import jax
import jax.numpy as jnp
from jax import lax
from jax.experimental import pallas as pl
from jax.experimental.pallas import tpu as pltpu
from jax.experimental.pallas import tpu_sc as plsc

VOCAB = 100000
HIDDEN = 128
BATCH = 4096
SEQ = 50
SEQ_PAD = 56
EPS = 1e-3

NUM_WORKERS = 32
B_SC = 2560
B_TC = BATCH - B_SC
ROWS_PER_WORKER = B_SC // NUM_WORKERS
LANES = 16
NCHUNK = HIDDEN // LANES
TC_CHUNK = 128


def _sc_pool_body(idx_hbm, table_hbm, out_hbm, idx_v, g0, g1, pooled, sem0, sem1):
    nc = 2
    wid = lax.axis_index("s") * nc + lax.axis_index("c")
    base = wid * ROWS_PER_WORKER

    pltpu.sync_copy(idx_hbm.at[pl.ds(base, ROWS_PER_WORKER)], idx_v)

    def gather(b, buf, sem):
        return pltpu.make_async_copy(
            table_hbm.at[idx_v.at[b, pl.ds(0, SEQ)]], buf, sem
        )

    def accumulate(buf, b):
        accs = [buf[0, pl.ds(c * LANES, LANES)] for c in range(NCHUNK)]
        for l in range(1, SEQ):
            for c in range(NCHUNK):
                accs[c] = accs[c] + buf[l, pl.ds(c * LANES, LANES)]
        for c in range(NCHUNK):
            pooled[b, pl.ds(c * LANES, LANES)] = accs[c] * (1.0 / SEQ)

    gather(0, g0, sem0).start()

    def loop_body(i, _):
        b = 2 * i
        gather(b, g0, sem0).wait()
        gather(b + 1, g1, sem1).start()
        accumulate(g0, b)
        gather(b + 1, g1, sem1).wait()

        @pl.when(b + 2 < ROWS_PER_WORKER)
        def _():
            gather(b + 2, g0, sem0).start()

        accumulate(g1, b + 1)
        return _

    lax.fori_loop(0, ROWS_PER_WORKER // 2, loop_body, None)

    pltpu.sync_copy(pooled, out_hbm.at[pl.ds(base, ROWS_PER_WORKER)])


@jax.jit
def _sc_pool(idx_padded, table):
    mesh = plsc.VectorSubcoreMesh(core_axis_name="c", subcore_axis_name="s")
    return pl.kernel(
        _sc_pool_body,
        mesh=mesh,
        out_type=jax.ShapeDtypeStruct((B_SC, HIDDEN), jnp.float32),
        scratch_types=[
            pltpu.VMEM((ROWS_PER_WORKER, SEQ_PAD), jnp.int32),
            pltpu.VMEM((SEQ, HIDDEN), jnp.float32),
            pltpu.VMEM((SEQ, HIDDEN), jnp.float32),
            pltpu.VMEM((ROWS_PER_WORKER, HIDDEN), jnp.float32),
            pltpu.SemaphoreType.DMA,
            pltpu.SemaphoreType.DMA,
        ],
    )(idx_padded, table)


N_COPY = 4
COPY_ROWS = VOCAB // N_COPY


def _tc_pool_body(idx_ref, table_hbm, out_ref, tvmem, *sems):
    @pl.when(pl.program_id(0) == 0)
    def _():
        for k in range(N_COPY):
            pltpu.make_async_copy(
                table_hbm.at[pl.ds(k * COPY_ROWS, COPY_ROWS)],
                tvmem.at[pl.ds(k * COPY_ROWS, COPY_ROWS)],
                sems[k],
            ).start()
        for k in range(N_COPY):
            pltpu.make_async_copy(
                table_hbm.at[pl.ds(k * COPY_ROWS, COPY_ROWS)],
                tvmem.at[pl.ds(k * COPY_ROWS, COPY_ROWS)],
                sems[k],
            ).wait()

    for r in range(TC_CHUNK):
        acc = tvmem[pl.ds(idx_ref[r, 0], 1), :]
        for l in range(1, SEQ):
            acc = acc + tvmem[pl.ds(idx_ref[r, l], 1), :]
        out_ref[pl.ds(r, 1), :] = acc * (1.0 / SEQ)


@jax.jit
def _tc_pool(idx, table):
    return pl.pallas_call(
        _tc_pool_body,
        grid=(B_TC // TC_CHUNK,),
        in_specs=[
            pl.BlockSpec((TC_CHUNK, SEQ), lambda i: (i, 0), memory_space=pltpu.SMEM),
            pl.BlockSpec(memory_space=pl.ANY),
        ],
        out_specs=pl.BlockSpec((TC_CHUNK, HIDDEN), lambda i: (i, 0)),
        out_shape=jax.ShapeDtypeStruct((B_TC, HIDDEN), jnp.float32),
        scratch_shapes=[pltpu.VMEM((VOCAB, HIDDEN), jnp.float32)]
        + [pltpu.SemaphoreType.DMA for _ in range(N_COPY)],
    )(idx, table)


def _tc_layernorm_body(a_ref, b_ref, gamma_ref, beta_ref, o_ref):
    a = a_ref[:, :]
    b = b_ref[:, :]
    mu = (jnp.sum(a, axis=0, keepdims=True) + jnp.sum(b, axis=0, keepdims=True)) * (
        1.0 / BATCH
    )
    da = a - mu
    db = b - mu
    var = (
        jnp.sum(da * da, axis=0, keepdims=True)
        + jnp.sum(db * db, axis=0, keepdims=True)
    ) * (1.0 / BATCH)
    r = lax.rsqrt(var + EPS)
    o_ref[pl.ds(0, B_TC), :] = (
        da * r * gamma_ref[pl.ds(0, B_TC), :] + beta_ref[pl.ds(0, B_TC), :]
    )
    o_ref[pl.ds(B_TC, B_SC), :] = (
        db * r * gamma_ref[pl.ds(B_TC, B_SC), :] + beta_ref[pl.ds(B_TC, B_SC), :]
    )


@jax.jit
def _tc_layernorm(pooled_tc, pooled_sc, gamma, beta):
    return pl.pallas_call(
        _tc_layernorm_body,
        out_shape=jax.ShapeDtypeStruct((BATCH, HIDDEN), jnp.float32),
    )(
        pooled_tc,
        pooled_sc,
        gamma.reshape(BATCH, 1),
        beta.reshape(BATCH, 1),
    )


def kernel(inputs, table, bias, gamma, beta):
    del bias
    idx_sc = jnp.concatenate(
        [inputs[B_TC:], jnp.zeros((B_SC, SEQ_PAD - SEQ), jnp.int32)], axis=1
    )
    pooled_sc = _sc_pool(idx_sc, table)
    pooled_tc = _tc_pool(inputs, table)
    return _tc_layernorm(pooled_tc, pooled_sc, gamma, beta)
